# baseline (speedup 1.0000x reference)
import functools

import jax
import jax.numpy as jnp
from jax import lax
from jax.experimental import pallas as pl
from jax.experimental.pallas import tpu as pltpu
from jax.experimental.pallas import tpu_sc as plsc

N = 10000
NPAD = 10240
E = 160000
DIN = 256
H = 128
OUT = 64
ALPHA = 0.5

NC = 2
NS = 16
NW = NC * NS
CHUNK = 128
CHUNKS_PER_W = 40
TOTC = NW * CHUNKS_PER_W
EPAD = TOTC * CHUNK
ROWS_PER_TILE = NPAD // NS
CA = 56
CB = 80 - CA
CMAX = max(CA, CB)

_mesh = plsc.VectorSubcoreMesh(core_axis_name="c", subcore_axis_name="s")


def _deg_body(dst_hbm, zeros_hbm, ones_hbm, out_hbm, idx_v, ones_v, acc_sh, sem):
    c = lax.axis_index("c")
    s = lax.axis_index("s")
    wid = c * NS + s
    pltpu.sync_copy(zeros_hbm, acc_sh.at[pl.ds(s * ROWS_PER_TILE, ROWS_PER_TILE)])
    pltpu.sync_copy(dst_hbm.at[pl.ds(wid * CHUNKS_PER_W, CHUNKS_PER_W)], idx_v)
    pltpu.sync_copy(ones_hbm, ones_v)
    plsc.subcore_barrier()

    @pl.loop(0, CHUNKS_PER_W)
    def _(j):
        pltpu.sync_copy(ones_v, acc_sh.at[idx_v.at[j]], add=True)

    plsc.subcore_barrier()
    pltpu.sync_copy(acc_sh.at[pl.ds(s * ROWS_PER_TILE, ROWS_PER_TILE)],
                    out_hbm.at[c, pl.ds(s * ROWS_PER_TILE, ROWS_PER_TILE)])


_deg_call = pl.kernel(
    _deg_body,
    out_type=jax.ShapeDtypeStruct((NC, NPAD), jnp.float32),
    mesh=_mesh,
    scratch_types=[
        pltpu.VMEM((CHUNKS_PER_W, CHUNK), jnp.int32),
        pltpu.VMEM((CHUNK,), jnp.float32),
        pltpu.VMEM_SHARED((NPAD,), jnp.float32),
        pltpu.SemaphoreType.DMA,
    ],
)


def _prop_pipeline(n, y_hbm, src_v, dst_v, rows0_v, rows1_v, acc_sh, sem0, sem1):
    pltpu.async_copy(y_hbm.at[src_v.at[0]], rows0_v, sem0)

    @pl.loop(0, n - 2, step=2)
    def _(j):
        pltpu.async_copy(y_hbm.at[src_v.at[j + 1]], rows1_v, sem1)
        pltpu.make_async_copy(y_hbm.at[src_v.at[j]], rows0_v, sem0).wait()
        pltpu.sync_copy(rows0_v, acc_sh.at[dst_v.at[j]], add=True)
        pltpu.async_copy(y_hbm.at[src_v.at[j + 2]], rows0_v, sem0)
        pltpu.make_async_copy(y_hbm.at[src_v.at[j + 1]], rows1_v, sem1).wait()
        pltpu.sync_copy(rows1_v, acc_sh.at[dst_v.at[j + 1]], add=True)

    _J = n - 2
    pltpu.async_copy(y_hbm.at[src_v.at[_J + 1]], rows1_v, sem1)
    pltpu.make_async_copy(y_hbm.at[src_v.at[_J]], rows0_v, sem0).wait()
    pltpu.sync_copy(rows0_v, acc_sh.at[dst_v.at[_J]], add=True)
    pltpu.make_async_copy(y_hbm.at[src_v.at[_J + 1]], rows1_v, sem1).wait()
    pltpu.sync_copy(rows1_v, acc_sh.at[dst_v.at[_J + 1]], add=True)


def _prop_body(y_hbm, src_hbm, dst_hbm, zeros_hbm, out_hbm,
               src_v, dst_v, rows0_v, rows1_v, acc_sh, sem0, sem1):
    c = lax.axis_index("c")
    s = lax.axis_index("s")
    base = jnp.where(c == 0, s * CA, NS * CA + s * CB)
    pltpu.sync_copy(zeros_hbm, acc_sh.at[pl.ds(s * ROWS_PER_TILE, ROWS_PER_TILE)])
    pltpu.sync_copy(src_hbm.at[pl.ds(base, CMAX)], src_v)
    pltpu.sync_copy(dst_hbm.at[pl.ds(base, CMAX)], dst_v)
    plsc.subcore_barrier()

    args = (y_hbm, src_v, dst_v, rows0_v, rows1_v, acc_sh, sem0, sem1)
    if CA == CB:
        _prop_pipeline(CA, *args)
    else:
        @pl.when(c == 0)
        def _():
            _prop_pipeline(CA, *args)

        @pl.when(c == 1)
        def _():
            _prop_pipeline(CB, *args)

    plsc.subcore_barrier()
    pltpu.sync_copy(acc_sh.at[pl.ds(s * ROWS_PER_TILE, ROWS_PER_TILE)],
                    out_hbm.at[c, pl.ds(s * ROWS_PER_TILE, ROWS_PER_TILE)])


_prop_call = pl.kernel(
    _prop_body,
    out_type=jax.ShapeDtypeStruct((NC, NPAD, H), jnp.float32),
    mesh=_mesh,
    scratch_types=[
        pltpu.VMEM((CMAX, CHUNK), jnp.int32),
        pltpu.VMEM((CMAX, CHUNK), jnp.int32),
        pltpu.VMEM((CHUNK, H), jnp.float32),
        pltpu.VMEM((CHUNK, H), jnp.float32),
        pltpu.VMEM_SHARED((NPAD, H), jnp.float32),
        pltpu.SemaphoreType.DMA,
        pltpu.SemaphoreType.DMA,
    ],
)


BR = 512
GRID = NPAD // BR


def _ln(u, g, b):
    mu = jnp.mean(u, axis=-1, keepdims=True)
    var = jnp.mean((u - mu) ** 2, axis=-1, keepdims=True)
    return (u - mu) * lax.rsqrt(var + 1e-5) * g + b


def _tc1_body(x_ref, wi_ref, bi_ref, dega_ref, degb_ref,
              x0_ref, y0_ref, dinv_ref):
    x0 = jnp.maximum(
        jnp.dot(x_ref[...], wi_ref[...], preferred_element_type=jnp.float32)
        + bi_ref[...], 0.0)
    deg = dega_ref[...] + degb_ref[...] + 1.0
    dinv = lax.rsqrt(deg)
    x0_ref[...] = x0
    y0_ref[...] = dinv * x0
    dinv_ref[...] = dinv


_tc1_call = pl.pallas_call(
    _tc1_body,
    grid=(GRID,),
    in_specs=[
        pl.BlockSpec((BR, DIN), lambda i: (i, 0)),
        pl.BlockSpec((DIN, H), lambda i: (0, 0)),
        pl.BlockSpec((1, H), lambda i: (0, 0)),
        pl.BlockSpec((BR, 1), lambda i: (i, 0)),
        pl.BlockSpec((BR, 1), lambda i: (i, 0)),
    ],
    out_specs=[
        pl.BlockSpec((BR, H), lambda i: (i, 0)),
        pl.BlockSpec((BR, H), lambda i: (i, 0)),
        pl.BlockSpec((BR, 1), lambda i: (i, 0)),
    ],
    out_shape=[
        jax.ShapeDtypeStruct((NPAD, H), jnp.float32),
        jax.ShapeDtypeStruct((NPAD, H), jnp.float32),
        jax.ShapeDtypeStruct((NPAD, 1), jnp.float32),
    ],
)


def _tc2_body(a0_ref, a1_ref, x0_ref, dinv_ref, w1_ref, g1_ref, be1_ref,
              h1_ref, y1_ref):
    dv = dinv_ref[...]
    x0 = x0_ref[...]
    h = dv * (a0_ref[...] + a1_ref[...]) + (dv * dv) * x0
    t = (1.0 - ALPHA) * h + ALPHA * x0
    u = jnp.dot(t, w1_ref[...], preferred_element_type=jnp.float32)
    h1 = jnp.maximum(_ln(u, g1_ref[...], be1_ref[...]), 0.0)
    h1_ref[...] = h1
    y1_ref[...] = dv * h1


_tc2_call = pl.pallas_call(
    _tc2_body,
    grid=(GRID,),
    in_specs=[
        pl.BlockSpec((BR, H), lambda i: (i, 0)),
        pl.BlockSpec((BR, H), lambda i: (i, 0)),
        pl.BlockSpec((BR, H), lambda i: (i, 0)),
        pl.BlockSpec((BR, 1), lambda i: (i, 0)),
        pl.BlockSpec((H, H), lambda i: (0, 0)),
        pl.BlockSpec((1, H), lambda i: (0, 0)),
        pl.BlockSpec((1, H), lambda i: (0, 0)),
    ],
    out_specs=[
        pl.BlockSpec((BR, H), lambda i: (i, 0)),
        pl.BlockSpec((BR, H), lambda i: (i, 0)),
    ],
    out_shape=[
        jax.ShapeDtypeStruct((NPAD, H), jnp.float32),
        jax.ShapeDtypeStruct((NPAD, H), jnp.float32),
    ],
)


def _tc3_body(a0_ref, a1_ref, h1_ref, x0_ref, dinv_ref, w2_ref, g2_ref,
              be2_ref, wr1_ref, br1_ref, gr_ref, ber_ref, wr2_ref, br2_ref,
              o_ref):
    dv = dinv_ref[...]
    h1 = h1_ref[...]
    h = dv * (a0_ref[...] + a1_ref[...]) + (dv * dv) * h1
    t = (1.0 - ALPHA) * h + ALPHA * x0_ref[...]
    u = jnp.dot(t, w2_ref[...], preferred_element_type=jnp.float32)
    h2 = jnp.maximum(_ln(u, g2_ref[...], be2_ref[...]), 0.0)
    z = jnp.dot(h2, wr1_ref[...], preferred_element_type=jnp.float32) + br1_ref[...]
    z = jnp.maximum(_ln(z, gr_ref[...], ber_ref[...]), 0.0)
    o_ref[...] = (jnp.dot(z, wr2_ref[...], preferred_element_type=jnp.float32)
                  + br2_ref[...])


_tc3_call = pl.pallas_call(
    _tc3_body,
    grid=(GRID,),
    in_specs=[
        pl.BlockSpec((BR, H), lambda i: (i, 0)),
        pl.BlockSpec((BR, H), lambda i: (i, 0)),
        pl.BlockSpec((BR, H), lambda i: (i, 0)),
        pl.BlockSpec((BR, H), lambda i: (i, 0)),
        pl.BlockSpec((BR, 1), lambda i: (i, 0)),
        pl.BlockSpec((H, H), lambda i: (0, 0)),
        pl.BlockSpec((1, H), lambda i: (0, 0)),
        pl.BlockSpec((1, H), lambda i: (0, 0)),
        pl.BlockSpec((H, 32), lambda i: (0, 0)),
        pl.BlockSpec((1, 32), lambda i: (0, 0)),
        pl.BlockSpec((1, 32), lambda i: (0, 0)),
        pl.BlockSpec((1, 32), lambda i: (0, 0)),
        pl.BlockSpec((32, OUT), lambda i: (0, 0)),
        pl.BlockSpec((1, OUT), lambda i: (0, 0)),
    ],
    out_specs=pl.BlockSpec((BR, OUT), lambda i: (i, 0)),
    out_shape=jax.ShapeDtypeStruct((NPAD, OUT), jnp.float32),
)


def kernel(x, edge_index, edge_weight, W_in, b_in, W1, g1, be1, W2, g2, be2,
           Wr1, br1, gr, ber, Wr2, br2):
    pad = (TOTC + CMAX) * CHUNK - E
    src_p = jnp.concatenate(
        [edge_index[0], jnp.zeros((pad,), jnp.int32)]).reshape(TOTC + CMAX, CHUNK)
    dst_p = jnp.concatenate(
        [edge_index[1], jnp.full((pad,), N, jnp.int32)]).reshape(TOTC + CMAX, CHUNK)
    x_pad = jnp.pad(x, ((0, NPAD - N), (0, 0)))

    z_deg = jnp.zeros((ROWS_PER_TILE,), jnp.float32)
    z_rows = jnp.zeros((ROWS_PER_TILE, H), jnp.float32)
    ones_c = jnp.ones((CHUNK,), jnp.float32)

    degp = _deg_call(dst_p, z_deg, ones_c)
    x0, y0, dinv = _tc1_call(x_pad, W_in, b_in[None], degp[0][:, None],
                             degp[1][:, None])

    acc1 = _prop_call(y0, src_p, dst_p, z_rows)
    h1, y1 = _tc2_call(acc1[0], acc1[1], x0, dinv, W1, g1[None], be1[None])

    acc2 = _prop_call(y1, src_p, dst_p, z_rows)
    out = _tc3_call(acc2[0], acc2[1], h1, x0, dinv, W2, g2[None], be2[None],
                    Wr1, br1[None], gr[None], ber[None], Wr2, br2[None])
    return out[:N]


# speedup vs baseline: 1.0025x; 1.0025x over previous
import functools

import jax
import jax.numpy as jnp
from jax import lax
from jax.experimental import pallas as pl
from jax.experimental.pallas import tpu as pltpu
from jax.experimental.pallas import tpu_sc as plsc

N = 10000
NPAD = 10240
E = 160000
DIN = 256
H = 128
OUT = 64
ALPHA = 0.5

NC = 2
NS = 16
NW = NC * NS
CHUNK = 128
CHUNKS_PER_W = 40
TOTC = NW * CHUNKS_PER_W
EPAD = TOTC * CHUNK
ROWS_PER_TILE = NPAD // NS
CA = 64
CB = 80 - CA
CMAX = max(CA, CB)

_mesh = plsc.VectorSubcoreMesh(core_axis_name="c", subcore_axis_name="s")


def _deg_body(dst_hbm, zeros_hbm, ones_hbm, out_hbm, idx_v, ones_v, acc_sh, sem):
    c = lax.axis_index("c")
    s = lax.axis_index("s")
    wid = c * NS + s
    pltpu.sync_copy(zeros_hbm, acc_sh.at[pl.ds(s * ROWS_PER_TILE, ROWS_PER_TILE)])
    pltpu.sync_copy(dst_hbm.at[pl.ds(wid * CHUNKS_PER_W, CHUNKS_PER_W)], idx_v)
    pltpu.sync_copy(ones_hbm, ones_v)
    plsc.subcore_barrier()

    @pl.loop(0, CHUNKS_PER_W)
    def _(j):
        pltpu.sync_copy(ones_v, acc_sh.at[idx_v.at[j]], add=True)

    plsc.subcore_barrier()
    pltpu.sync_copy(acc_sh.at[pl.ds(s * ROWS_PER_TILE, ROWS_PER_TILE)],
                    out_hbm.at[c, pl.ds(s * ROWS_PER_TILE, ROWS_PER_TILE)])


_deg_call = pl.kernel(
    _deg_body,
    out_type=jax.ShapeDtypeStruct((NC, NPAD), jnp.float32),
    mesh=_mesh,
    scratch_types=[
        pltpu.VMEM((CHUNKS_PER_W, CHUNK), jnp.int32),
        pltpu.VMEM((CHUNK,), jnp.float32),
        pltpu.VMEM_SHARED((NPAD,), jnp.float32),
        pltpu.SemaphoreType.DMA,
    ],
)


_NSPLIT = 4
_SPL = CHUNK // _NSPLIT


def _prop_pipeline(n, y_hbm, src_v, dst_v, rows0_v, rows1_v, acc_sh, sem0, sem1):
    def _gather(j, rows, sem):
        for p in range(_NSPLIT):
            pltpu.async_copy(y_hbm.at[src_v.at[j, pl.ds(p * _SPL, _SPL)]],
                             rows.at[pl.ds(p * _SPL, _SPL)], sem)

    def _gwait(j, rows, sem):
        for p in range(_NSPLIT):
            pltpu.make_async_copy(y_hbm.at[src_v.at[j, pl.ds(p * _SPL, _SPL)]],
                                  rows.at[pl.ds(p * _SPL, _SPL)], sem).wait()

    _gather(0, rows0_v, sem0)

    @pl.loop(0, n - 2, step=2)
    def _(j):
        _gather(j + 1, rows1_v, sem1)
        _gwait(j, rows0_v, sem0)
        pltpu.sync_copy(rows0_v, acc_sh.at[dst_v.at[j]], add=True)
        _gather(j + 2, rows0_v, sem0)
        _gwait(j + 1, rows1_v, sem1)
        pltpu.sync_copy(rows1_v, acc_sh.at[dst_v.at[j + 1]], add=True)

    _J = n - 2
    _gather(_J + 1, rows1_v, sem1)
    _gwait(_J, rows0_v, sem0)
    pltpu.sync_copy(rows0_v, acc_sh.at[dst_v.at[_J]], add=True)
    _gwait(_J + 1, rows1_v, sem1)
    pltpu.sync_copy(rows1_v, acc_sh.at[dst_v.at[_J + 1]], add=True)


def _prop_body(y_hbm, src_hbm, dst_hbm, zeros_hbm, out_hbm,
               src_v, dst_v, rows0_v, rows1_v, acc_sh, sem0, sem1):
    c = lax.axis_index("c")
    s = lax.axis_index("s")
    base = jnp.where(c == 0, s * CA, NS * CA + s * CB)
    pltpu.sync_copy(zeros_hbm, acc_sh.at[pl.ds(s * ROWS_PER_TILE, ROWS_PER_TILE)])
    pltpu.sync_copy(src_hbm.at[pl.ds(base, CMAX)], src_v)
    pltpu.sync_copy(dst_hbm.at[pl.ds(base, CMAX)], dst_v)
    plsc.subcore_barrier()

    args = (y_hbm, src_v, dst_v, rows0_v, rows1_v, acc_sh, sem0, sem1)
    if CA == CB:
        _prop_pipeline(CA, *args)
    else:
        @pl.when(c == 0)
        def _():
            _prop_pipeline(CA, *args)

        @pl.when(c == 1)
        def _():
            _prop_pipeline(CB, *args)

    plsc.subcore_barrier()
    pltpu.sync_copy(acc_sh.at[pl.ds(s * ROWS_PER_TILE, ROWS_PER_TILE)],
                    out_hbm.at[c, pl.ds(s * ROWS_PER_TILE, ROWS_PER_TILE)])


_prop_call = pl.kernel(
    _prop_body,
    out_type=jax.ShapeDtypeStruct((NC, NPAD, H), jnp.float32),
    mesh=_mesh,
    scratch_types=[
        pltpu.VMEM((CMAX, CHUNK), jnp.int32),
        pltpu.VMEM((CMAX, CHUNK), jnp.int32),
        pltpu.VMEM((CHUNK, H), jnp.float32),
        pltpu.VMEM((CHUNK, H), jnp.float32),
        pltpu.VMEM_SHARED((NPAD, H), jnp.float32),
        pltpu.SemaphoreType.DMA,
        pltpu.SemaphoreType.DMA,
    ],
)


BR = 512
GRID = NPAD // BR


def _ln(u, g, b):
    mu = jnp.mean(u, axis=-1, keepdims=True)
    var = jnp.mean((u - mu) ** 2, axis=-1, keepdims=True)
    return (u - mu) * lax.rsqrt(var + 1e-5) * g + b


def _tc1_body(x_ref, wi_ref, bi_ref, dega_ref, degb_ref,
              x0_ref, y0_ref, dinv_ref):
    x0 = jnp.maximum(
        jnp.dot(x_ref[...], wi_ref[...], preferred_element_type=jnp.float32)
        + bi_ref[...], 0.0)
    deg = dega_ref[...] + degb_ref[...] + 1.0
    dinv = lax.rsqrt(deg)
    x0_ref[...] = x0
    y0_ref[...] = dinv * x0
    dinv_ref[...] = dinv


_tc1_call = pl.pallas_call(
    _tc1_body,
    grid=(GRID,),
    in_specs=[
        pl.BlockSpec((BR, DIN), lambda i: (i, 0)),
        pl.BlockSpec((DIN, H), lambda i: (0, 0)),
        pl.BlockSpec((1, H), lambda i: (0, 0)),
        pl.BlockSpec((BR, 1), lambda i: (i, 0)),
        pl.BlockSpec((BR, 1), lambda i: (i, 0)),
    ],
    out_specs=[
        pl.BlockSpec((BR, H), lambda i: (i, 0)),
        pl.BlockSpec((BR, H), lambda i: (i, 0)),
        pl.BlockSpec((BR, 1), lambda i: (i, 0)),
    ],
    out_shape=[
        jax.ShapeDtypeStruct((NPAD, H), jnp.float32),
        jax.ShapeDtypeStruct((NPAD, H), jnp.float32),
        jax.ShapeDtypeStruct((NPAD, 1), jnp.float32),
    ],
)


def _tc2_body(a0_ref, a1_ref, x0_ref, dinv_ref, w1_ref, g1_ref, be1_ref,
              h1_ref, y1_ref):
    dv = dinv_ref[...]
    x0 = x0_ref[...]
    h = dv * (a0_ref[...] + a1_ref[...]) + (dv * dv) * x0
    t = (1.0 - ALPHA) * h + ALPHA * x0
    u = jnp.dot(t, w1_ref[...], preferred_element_type=jnp.float32)
    h1 = jnp.maximum(_ln(u, g1_ref[...], be1_ref[...]), 0.0)
    h1_ref[...] = h1
    y1_ref[...] = dv * h1


_tc2_call = pl.pallas_call(
    _tc2_body,
    grid=(GRID,),
    in_specs=[
        pl.BlockSpec((BR, H), lambda i: (i, 0)),
        pl.BlockSpec((BR, H), lambda i: (i, 0)),
        pl.BlockSpec((BR, H), lambda i: (i, 0)),
        pl.BlockSpec((BR, 1), lambda i: (i, 0)),
        pl.BlockSpec((H, H), lambda i: (0, 0)),
        pl.BlockSpec((1, H), lambda i: (0, 0)),
        pl.BlockSpec((1, H), lambda i: (0, 0)),
    ],
    out_specs=[
        pl.BlockSpec((BR, H), lambda i: (i, 0)),
        pl.BlockSpec((BR, H), lambda i: (i, 0)),
    ],
    out_shape=[
        jax.ShapeDtypeStruct((NPAD, H), jnp.float32),
        jax.ShapeDtypeStruct((NPAD, H), jnp.float32),
    ],
)


def _tc3_body(a0_ref, a1_ref, h1_ref, x0_ref, dinv_ref, w2_ref, g2_ref,
              be2_ref, wr1_ref, br1_ref, gr_ref, ber_ref, wr2_ref, br2_ref,
              o_ref):
    dv = dinv_ref[...]
    h1 = h1_ref[...]
    h = dv * (a0_ref[...] + a1_ref[...]) + (dv * dv) * h1
    t = (1.0 - ALPHA) * h + ALPHA * x0_ref[...]
    u = jnp.dot(t, w2_ref[...], preferred_element_type=jnp.float32)
    h2 = jnp.maximum(_ln(u, g2_ref[...], be2_ref[...]), 0.0)
    z = jnp.dot(h2, wr1_ref[...], preferred_element_type=jnp.float32) + br1_ref[...]
    z = jnp.maximum(_ln(z, gr_ref[...], ber_ref[...]), 0.0)
    o_ref[...] = (jnp.dot(z, wr2_ref[...], preferred_element_type=jnp.float32)
                  + br2_ref[...])


_tc3_call = pl.pallas_call(
    _tc3_body,
    grid=(GRID,),
    in_specs=[
        pl.BlockSpec((BR, H), lambda i: (i, 0)),
        pl.BlockSpec((BR, H), lambda i: (i, 0)),
        pl.BlockSpec((BR, H), lambda i: (i, 0)),
        pl.BlockSpec((BR, H), lambda i: (i, 0)),
        pl.BlockSpec((BR, 1), lambda i: (i, 0)),
        pl.BlockSpec((H, H), lambda i: (0, 0)),
        pl.BlockSpec((1, H), lambda i: (0, 0)),
        pl.BlockSpec((1, H), lambda i: (0, 0)),
        pl.BlockSpec((H, 32), lambda i: (0, 0)),
        pl.BlockSpec((1, 32), lambda i: (0, 0)),
        pl.BlockSpec((1, 32), lambda i: (0, 0)),
        pl.BlockSpec((1, 32), lambda i: (0, 0)),
        pl.BlockSpec((32, OUT), lambda i: (0, 0)),
        pl.BlockSpec((1, OUT), lambda i: (0, 0)),
    ],
    out_specs=pl.BlockSpec((BR, OUT), lambda i: (i, 0)),
    out_shape=jax.ShapeDtypeStruct((NPAD, OUT), jnp.float32),
)


def kernel(x, edge_index, edge_weight, W_in, b_in, W1, g1, be1, W2, g2, be2,
           Wr1, br1, gr, ber, Wr2, br2):
    pad = (TOTC + CMAX) * CHUNK - E
    src_p = jnp.concatenate(
        [edge_index[0], jnp.zeros((pad,), jnp.int32)]).reshape(TOTC + CMAX, CHUNK)
    dst_p = jnp.concatenate(
        [edge_index[1], jnp.full((pad,), N, jnp.int32)]).reshape(TOTC + CMAX, CHUNK)
    x_pad = jnp.pad(x, ((0, NPAD - N), (0, 0)))

    z_deg = jnp.zeros((ROWS_PER_TILE,), jnp.float32)
    z_rows = jnp.zeros((ROWS_PER_TILE, H), jnp.float32)
    ones_c = jnp.ones((CHUNK,), jnp.float32)

    degp = _deg_call(dst_p, z_deg, ones_c)
    x0, y0, dinv = _tc1_call(x_pad, W_in, b_in[None], degp[0][:, None],
                             degp[1][:, None])

    acc1 = _prop_call(y0, src_p, dst_p, z_rows)
    h1, y1 = _tc2_call(acc1[0], acc1[1], x0, dinv, W1, g1[None], be1[None])

    acc2 = _prop_call(y1, src_p, dst_p, z_rows)
    out = _tc3_call(acc2[0], acc2[1], h1, x0, dinv, W2, g2[None], be2[None],
                    Wr1, br1[None], gr[None], ber[None], Wr2, br2[None])
    return out[:N]


# speedup vs baseline: 1.0482x; 1.0456x over previous
import functools

import jax
import jax.numpy as jnp
from jax import lax
from jax.experimental import pallas as pl
from jax.experimental.pallas import tpu as pltpu
from jax.experimental.pallas import tpu_sc as plsc

N = 10000
NPAD = 10240
E = 160000
DIN = 256
H = 128
OUT = 64
ALPHA = 0.5

NC = 2
NS = 16
NW = NC * NS
CHUNK = 128
CHUNKS_PER_W = 40
TOTC = NW * CHUNKS_PER_W
EPAD = TOTC * CHUNK
ROWS_PER_TILE = NPAD // NS
CA = 64
CB = 80 - CA
CMAX = max(CA, CB)

_mesh = plsc.VectorSubcoreMesh(core_axis_name="c", subcore_axis_name="s")


def _deg_body(dst_hbm, zeros_hbm, ones_hbm, out_hbm, idx_v, ones_v, acc_sh, sem):
    c = lax.axis_index("c")
    s = lax.axis_index("s")
    wid = c * NS + s
    pltpu.sync_copy(zeros_hbm, acc_sh.at[pl.ds(s * ROWS_PER_TILE, ROWS_PER_TILE)])
    pltpu.sync_copy(dst_hbm.at[pl.ds(wid * CHUNKS_PER_W, CHUNKS_PER_W)], idx_v)
    pltpu.sync_copy(ones_hbm, ones_v)
    plsc.subcore_barrier()

    @pl.loop(0, CHUNKS_PER_W)
    def _(j):
        pltpu.sync_copy(ones_v, acc_sh.at[idx_v.at[j]], add=True)

    plsc.subcore_barrier()
    pltpu.sync_copy(acc_sh.at[pl.ds(s * ROWS_PER_TILE, ROWS_PER_TILE)],
                    out_hbm.at[c, pl.ds(s * ROWS_PER_TILE, ROWS_PER_TILE)])


_deg_call = pl.kernel(
    _deg_body,
    out_type=jax.ShapeDtypeStruct((NC, NPAD), jnp.float32),
    mesh=_mesh,
    scratch_types=[
        pltpu.VMEM((CHUNKS_PER_W, CHUNK), jnp.int32),
        pltpu.VMEM((CHUNK,), jnp.float32),
        pltpu.VMEM_SHARED((NPAD,), jnp.float32),
        pltpu.SemaphoreType.DMA,
    ],
)


_NSPLIT = 4
_SPL = CHUNK // _NSPLIT


def _prop_pipeline(n, y_hbm, src_v, dst_v, rows0_v, rows1_v, acc_sh, sem0, sem1):
    def _gather(j, rows, sem):
        for p in range(_NSPLIT):
            pltpu.async_copy(y_hbm.at[src_v.at[j, pl.ds(p * _SPL, _SPL)]],
                             rows.at[pl.ds(p * _SPL, _SPL)], sem)

    def _gwait(j, rows, sem):
        for p in range(_NSPLIT):
            pltpu.make_async_copy(y_hbm.at[src_v.at[j, pl.ds(p * _SPL, _SPL)]],
                                  rows.at[pl.ds(p * _SPL, _SPL)], sem).wait()

    _gather(0, rows0_v, sem0)

    @pl.loop(0, n - 2, step=2)
    def _(j):
        _gather(j + 1, rows1_v, sem1)
        _gwait(j, rows0_v, sem0)
        pltpu.sync_copy(rows0_v, acc_sh.at[dst_v.at[j]], add=True)
        _gather(j + 2, rows0_v, sem0)
        _gwait(j + 1, rows1_v, sem1)
        pltpu.sync_copy(rows1_v, acc_sh.at[dst_v.at[j + 1]], add=True)

    _J = n - 2
    _gather(_J + 1, rows1_v, sem1)
    _gwait(_J, rows0_v, sem0)
    pltpu.sync_copy(rows0_v, acc_sh.at[dst_v.at[_J]], add=True)
    _gwait(_J + 1, rows1_v, sem1)
    pltpu.sync_copy(rows1_v, acc_sh.at[dst_v.at[_J + 1]], add=True)


def _prop_body(y_hbm, src_hbm, dst_hbm, out_hbm,
               src_v, dst_v, rows0_v, rows1_v, acc_sh, sem0, sem1):
    c = lax.axis_index("c")
    s = lax.axis_index("s")
    base = jnp.where(c == 0, s * CA, NS * CA + s * CB)

    @pl.loop(0, CHUNK)
    def _(i):
        for k in range(H // 16):
            rows0_v[i, pl.ds(k * 16, 16)] = jnp.zeros((16,), jnp.float32)

    for t in range(ROWS_PER_TILE // CHUNK):
        pltpu.sync_copy(rows0_v,
                        acc_sh.at[pl.ds(s * ROWS_PER_TILE + t * CHUNK, CHUNK)])
    pltpu.sync_copy(src_hbm.at[pl.ds(base, CMAX)], src_v)
    pltpu.sync_copy(dst_hbm.at[pl.ds(base, CMAX)], dst_v)
    plsc.subcore_barrier()

    args = (y_hbm, src_v, dst_v, rows0_v, rows1_v, acc_sh, sem0, sem1)
    if CA == CB:
        _prop_pipeline(CA, *args)
    else:
        @pl.when(c == 0)
        def _():
            _prop_pipeline(CA, *args)

        @pl.when(c == 1)
        def _():
            _prop_pipeline(CB, *args)

    plsc.subcore_barrier()
    pltpu.sync_copy(acc_sh.at[pl.ds(s * ROWS_PER_TILE, ROWS_PER_TILE)],
                    out_hbm.at[c, pl.ds(s * ROWS_PER_TILE, ROWS_PER_TILE)])


_prop_call = pl.kernel(
    _prop_body,
    out_type=jax.ShapeDtypeStruct((NC, NPAD, H), jnp.float32),
    mesh=_mesh,
    scratch_types=[
        pltpu.VMEM((CMAX, CHUNK), jnp.int32),
        pltpu.VMEM((CMAX, CHUNK), jnp.int32),
        pltpu.VMEM((CHUNK, H), jnp.float32),
        pltpu.VMEM((CHUNK, H), jnp.float32),
        pltpu.VMEM_SHARED((NPAD, H), jnp.float32),
        pltpu.SemaphoreType.DMA,
        pltpu.SemaphoreType.DMA,
    ],
)


BR = 512
GRID = NPAD // BR


def _ln(u, g, b):
    mu = jnp.mean(u, axis=-1, keepdims=True)
    var = jnp.mean((u - mu) ** 2, axis=-1, keepdims=True)
    return (u - mu) * lax.rsqrt(var + 1e-5) * g + b


def _tc1_body(x_ref, wi_ref, bi_ref, dega_ref, degb_ref,
              x0_ref, y0_ref, dinv_ref):
    x0 = jnp.maximum(
        jnp.dot(x_ref[...], wi_ref[...], preferred_element_type=jnp.float32)
        + bi_ref[...], 0.0)
    deg = dega_ref[...] + degb_ref[...] + 1.0
    dinv = lax.rsqrt(deg)
    x0_ref[...] = x0
    y0_ref[...] = dinv * x0
    dinv_ref[...] = dinv


_tc1_call = pl.pallas_call(
    _tc1_body,
    grid=(GRID,),
    in_specs=[
        pl.BlockSpec((BR, DIN), lambda i: (i, 0)),
        pl.BlockSpec((DIN, H), lambda i: (0, 0)),
        pl.BlockSpec((1, H), lambda i: (0, 0)),
        pl.BlockSpec((BR, 1), lambda i: (i, 0)),
        pl.BlockSpec((BR, 1), lambda i: (i, 0)),
    ],
    out_specs=[
        pl.BlockSpec((BR, H), lambda i: (i, 0)),
        pl.BlockSpec((BR, H), lambda i: (i, 0)),
        pl.BlockSpec((BR, 1), lambda i: (i, 0)),
    ],
    out_shape=[
        jax.ShapeDtypeStruct((NPAD, H), jnp.float32),
        jax.ShapeDtypeStruct((NPAD, H), jnp.float32),
        jax.ShapeDtypeStruct((NPAD, 1), jnp.float32),
    ],
)


def _tc2_body(a0_ref, a1_ref, x0_ref, dinv_ref, w1_ref, g1_ref, be1_ref,
              h1_ref, y1_ref):
    dv = dinv_ref[...]
    x0 = x0_ref[...]
    h = dv * (a0_ref[...] + a1_ref[...]) + (dv * dv) * x0
    t = (1.0 - ALPHA) * h + ALPHA * x0
    u = jnp.dot(t, w1_ref[...], preferred_element_type=jnp.float32)
    h1 = jnp.maximum(_ln(u, g1_ref[...], be1_ref[...]), 0.0)
    h1_ref[...] = h1
    y1_ref[...] = dv * h1


_tc2_call = pl.pallas_call(
    _tc2_body,
    grid=(GRID,),
    in_specs=[
        pl.BlockSpec((BR, H), lambda i: (i, 0)),
        pl.BlockSpec((BR, H), lambda i: (i, 0)),
        pl.BlockSpec((BR, H), lambda i: (i, 0)),
        pl.BlockSpec((BR, 1), lambda i: (i, 0)),
        pl.BlockSpec((H, H), lambda i: (0, 0)),
        pl.BlockSpec((1, H), lambda i: (0, 0)),
        pl.BlockSpec((1, H), lambda i: (0, 0)),
    ],
    out_specs=[
        pl.BlockSpec((BR, H), lambda i: (i, 0)),
        pl.BlockSpec((BR, H), lambda i: (i, 0)),
    ],
    out_shape=[
        jax.ShapeDtypeStruct((NPAD, H), jnp.float32),
        jax.ShapeDtypeStruct((NPAD, H), jnp.float32),
    ],
)


def _tc3_body(a0_ref, a1_ref, h1_ref, x0_ref, dinv_ref, w2_ref, g2_ref,
              be2_ref, wr1_ref, br1_ref, gr_ref, ber_ref, wr2_ref, br2_ref,
              o_ref):
    dv = dinv_ref[...]
    h1 = h1_ref[...]
    h = dv * (a0_ref[...] + a1_ref[...]) + (dv * dv) * h1
    t = (1.0 - ALPHA) * h + ALPHA * x0_ref[...]
    u = jnp.dot(t, w2_ref[...], preferred_element_type=jnp.float32)
    h2 = jnp.maximum(_ln(u, g2_ref[...], be2_ref[...]), 0.0)
    z = jnp.dot(h2, wr1_ref[...], preferred_element_type=jnp.float32) + br1_ref[...]
    z = jnp.maximum(_ln(z, gr_ref[...], ber_ref[...]), 0.0)
    o_ref[...] = (jnp.dot(z, wr2_ref[...], preferred_element_type=jnp.float32)
                  + br2_ref[...])


_tc3_call = pl.pallas_call(
    _tc3_body,
    grid=(GRID,),
    in_specs=[
        pl.BlockSpec((BR, H), lambda i: (i, 0)),
        pl.BlockSpec((BR, H), lambda i: (i, 0)),
        pl.BlockSpec((BR, H), lambda i: (i, 0)),
        pl.BlockSpec((BR, H), lambda i: (i, 0)),
        pl.BlockSpec((BR, 1), lambda i: (i, 0)),
        pl.BlockSpec((H, H), lambda i: (0, 0)),
        pl.BlockSpec((1, H), lambda i: (0, 0)),
        pl.BlockSpec((1, H), lambda i: (0, 0)),
        pl.BlockSpec((H, 32), lambda i: (0, 0)),
        pl.BlockSpec((1, 32), lambda i: (0, 0)),
        pl.BlockSpec((1, 32), lambda i: (0, 0)),
        pl.BlockSpec((1, 32), lambda i: (0, 0)),
        pl.BlockSpec((32, OUT), lambda i: (0, 0)),
        pl.BlockSpec((1, OUT), lambda i: (0, 0)),
    ],
    out_specs=pl.BlockSpec((BR, OUT), lambda i: (i, 0)),
    out_shape=jax.ShapeDtypeStruct((NPAD, OUT), jnp.float32),
)


def kernel(x, edge_index, edge_weight, W_in, b_in, W1, g1, be1, W2, g2, be2,
           Wr1, br1, gr, ber, Wr2, br2):
    pad = (TOTC + CMAX) * CHUNK - E
    src_p = jnp.concatenate(
        [edge_index[0], jnp.zeros((pad,), jnp.int32)]).reshape(TOTC + CMAX, CHUNK)
    dst_p = jnp.concatenate(
        [edge_index[1], jnp.full((pad,), N, jnp.int32)]).reshape(TOTC + CMAX, CHUNK)
    x_pad = jnp.pad(x, ((0, NPAD - N), (0, 0)))

    z_deg = jnp.zeros((ROWS_PER_TILE,), jnp.float32)
    ones_c = jnp.ones((CHUNK,), jnp.float32)

    degp = _deg_call(dst_p, z_deg, ones_c)
    x0, y0, dinv = _tc1_call(x_pad, W_in, b_in[None], degp[0][:, None],
                             degp[1][:, None])

    acc1 = _prop_call(y0, src_p, dst_p)
    h1, y1 = _tc2_call(acc1[0], acc1[1], x0, dinv, W1, g1[None], be1[None])

    acc2 = _prop_call(y1, src_p, dst_p)
    out = _tc3_call(acc2[0], acc2[1], h1, x0, dinv, W2, g2[None], be2[None],
                    Wr1, br1[None], gr[None], ber[None], Wr2, br2[None])
    return out[:N]


# speedup vs baseline: 1.0600x; 1.0112x over previous
import functools

import jax
import jax.numpy as jnp
from jax import lax
from jax.experimental import pallas as pl
from jax.experimental.pallas import tpu as pltpu
from jax.experimental.pallas import tpu_sc as plsc

N = 10000
NPAD = 10240
E = 160000
DIN = 256
H = 128
OUT = 64
ALPHA = 0.5

NC = 2
NS = 16
NW = NC * NS
CHUNK = 128
CHUNKS_PER_W = 40
TOTC = NW * CHUNKS_PER_W
EPAD = TOTC * CHUNK
ROWS_PER_TILE = NPAD // NS
CA = 64
CB = 80 - CA
CMAX = max(CA, CB)

_mesh = plsc.VectorSubcoreMesh(core_axis_name="c", subcore_axis_name="s")


def _deg_body(dst_hbm, zeros_hbm, ones_hbm, out_hbm, idx_v, ones_v, acc_sh, sem):
    c = lax.axis_index("c")
    s = lax.axis_index("s")
    wid = c * NS + s
    pltpu.sync_copy(zeros_hbm, acc_sh.at[pl.ds(s * ROWS_PER_TILE, ROWS_PER_TILE)])
    pltpu.sync_copy(dst_hbm.at[pl.ds(wid * CHUNKS_PER_W, CHUNKS_PER_W)], idx_v)
    pltpu.sync_copy(ones_hbm, ones_v)
    plsc.subcore_barrier()

    @pl.loop(0, CHUNKS_PER_W)
    def _(j):
        pltpu.sync_copy(ones_v, acc_sh.at[idx_v.at[j]], add=True)

    plsc.subcore_barrier()
    pltpu.sync_copy(acc_sh.at[pl.ds(s * ROWS_PER_TILE, ROWS_PER_TILE)],
                    out_hbm.at[c, pl.ds(s * ROWS_PER_TILE, ROWS_PER_TILE)])


_deg_call = pl.kernel(
    _deg_body,
    out_type=jax.ShapeDtypeStruct((NC, NPAD), jnp.float32),
    mesh=_mesh,
    scratch_types=[
        pltpu.VMEM((CHUNKS_PER_W, CHUNK), jnp.int32),
        pltpu.VMEM((CHUNK,), jnp.float32),
        pltpu.VMEM_SHARED((NPAD,), jnp.float32),
        pltpu.SemaphoreType.DMA,
    ],
)


def _prop_pipeline(n, y_hbm, src_v, dst_v, rows0_v, rows1_v, acc_sh,
                   sem0, sem1, ssem0, ssem1):
    def _gather(j, rows, sem):
        pltpu.async_copy(y_hbm.at[src_v.at[j]], rows, sem)

    def _gwait(j, rows, sem):
        pltpu.make_async_copy(y_hbm.at[src_v.at[j]], rows, sem).wait()

    def _scat(j, rows, sem):
        pltpu.async_copy(rows, acc_sh.at[dst_v.at[j]], sem, add=True)

    def _swait(j, rows, sem):
        pltpu.make_async_copy(rows, acc_sh.at[dst_v.at[j]], sem).wait()

    _gather(0, rows0_v, sem0)
    _gather(1, rows1_v, sem1)

    @pl.loop(0, n - 2, step=2)
    def _(j):
        _gwait(j, rows0_v, sem0)
        _scat(j, rows0_v, ssem0)
        _gwait(j + 1, rows1_v, sem1)
        _scat(j + 1, rows1_v, ssem1)
        _swait(j, rows0_v, ssem0)
        _gather(j + 2, rows0_v, sem0)
        _swait(j + 1, rows1_v, ssem1)
        _gather(j + 3, rows1_v, sem1)

    _J = n - 2
    _gwait(_J, rows0_v, sem0)
    _scat(_J, rows0_v, ssem0)
    _gwait(_J + 1, rows1_v, sem1)
    _scat(_J + 1, rows1_v, ssem1)
    _swait(_J, rows0_v, ssem0)
    _swait(_J + 1, rows1_v, ssem1)


def _prop_body(y_hbm, src_hbm, dst_hbm, out_hbm,
               src_v, dst_v, rows0_v, rows1_v, acc_sh, sem0, sem1, ssem0, ssem1):
    c = lax.axis_index("c")
    s = lax.axis_index("s")
    base = jnp.where(c == 0, s * CA, NS * CA + s * CB)

    @pl.loop(0, CHUNK)
    def _(i):
        for k in range(H // 16):
            rows0_v[i, pl.ds(k * 16, 16)] = jnp.zeros((16,), jnp.float32)

    for t in range(ROWS_PER_TILE // CHUNK):
        pltpu.sync_copy(rows0_v,
                        acc_sh.at[pl.ds(s * ROWS_PER_TILE + t * CHUNK, CHUNK)])
    pltpu.sync_copy(src_hbm.at[pl.ds(base, CMAX)], src_v)
    pltpu.sync_copy(dst_hbm.at[pl.ds(base, CMAX)], dst_v)
    plsc.subcore_barrier()

    args = (y_hbm, src_v, dst_v, rows0_v, rows1_v, acc_sh, sem0, sem1,
            ssem0, ssem1)
    if CA == CB:
        _prop_pipeline(CA, *args)
    else:
        @pl.when(c == 0)
        def _():
            _prop_pipeline(CA, *args)

        @pl.when(c == 1)
        def _():
            _prop_pipeline(CB, *args)

    plsc.subcore_barrier()
    pltpu.sync_copy(acc_sh.at[pl.ds(s * ROWS_PER_TILE, ROWS_PER_TILE)],
                    out_hbm.at[c, pl.ds(s * ROWS_PER_TILE, ROWS_PER_TILE)])


_prop_call = pl.kernel(
    _prop_body,
    out_type=jax.ShapeDtypeStruct((NC, NPAD, H), jnp.float32),
    mesh=_mesh,
    scratch_types=[
        pltpu.VMEM((CMAX, CHUNK), jnp.int32),
        pltpu.VMEM((CMAX, CHUNK), jnp.int32),
        pltpu.VMEM((CHUNK, H), jnp.float32),
        pltpu.VMEM((CHUNK, H), jnp.float32),
        pltpu.VMEM_SHARED((NPAD, H), jnp.float32),
        pltpu.SemaphoreType.DMA,
        pltpu.SemaphoreType.DMA,
        pltpu.SemaphoreType.DMA,
        pltpu.SemaphoreType.DMA,
    ],
)


BR = 512
GRID = NPAD // BR


def _ln(u, g, b):
    mu = jnp.mean(u, axis=-1, keepdims=True)
    var = jnp.mean((u - mu) ** 2, axis=-1, keepdims=True)
    return (u - mu) * lax.rsqrt(var + 1e-5) * g + b


def _tc1_body(x_ref, wi_ref, bi_ref, dega_ref, degb_ref,
              x0_ref, y0_ref, dinv_ref):
    x0 = jnp.maximum(
        jnp.dot(x_ref[...], wi_ref[...], preferred_element_type=jnp.float32)
        + bi_ref[...], 0.0)
    deg = dega_ref[...] + degb_ref[...] + 1.0
    dinv = lax.rsqrt(deg)
    x0_ref[...] = x0
    y0_ref[...] = dinv * x0
    dinv_ref[...] = dinv


_tc1_call = pl.pallas_call(
    _tc1_body,
    grid=(GRID,),
    in_specs=[
        pl.BlockSpec((BR, DIN), lambda i: (i, 0)),
        pl.BlockSpec((DIN, H), lambda i: (0, 0)),
        pl.BlockSpec((1, H), lambda i: (0, 0)),
        pl.BlockSpec((BR, 1), lambda i: (i, 0)),
        pl.BlockSpec((BR, 1), lambda i: (i, 0)),
    ],
    out_specs=[
        pl.BlockSpec((BR, H), lambda i: (i, 0)),
        pl.BlockSpec((BR, H), lambda i: (i, 0)),
        pl.BlockSpec((BR, 1), lambda i: (i, 0)),
    ],
    out_shape=[
        jax.ShapeDtypeStruct((NPAD, H), jnp.float32),
        jax.ShapeDtypeStruct((NPAD, H), jnp.float32),
        jax.ShapeDtypeStruct((NPAD, 1), jnp.float32),
    ],
)


def _tc2_body(a0_ref, a1_ref, x0_ref, dinv_ref, w1_ref, g1_ref, be1_ref,
              h1_ref, y1_ref):
    dv = dinv_ref[...]
    x0 = x0_ref[...]
    h = dv * (a0_ref[...] + a1_ref[...]) + (dv * dv) * x0
    t = (1.0 - ALPHA) * h + ALPHA * x0
    u = jnp.dot(t, w1_ref[...], preferred_element_type=jnp.float32)
    h1 = jnp.maximum(_ln(u, g1_ref[...], be1_ref[...]), 0.0)
    h1_ref[...] = h1
    y1_ref[...] = dv * h1


_tc2_call = pl.pallas_call(
    _tc2_body,
    grid=(GRID,),
    in_specs=[
        pl.BlockSpec((BR, H), lambda i: (i, 0)),
        pl.BlockSpec((BR, H), lambda i: (i, 0)),
        pl.BlockSpec((BR, H), lambda i: (i, 0)),
        pl.BlockSpec((BR, 1), lambda i: (i, 0)),
        pl.BlockSpec((H, H), lambda i: (0, 0)),
        pl.BlockSpec((1, H), lambda i: (0, 0)),
        pl.BlockSpec((1, H), lambda i: (0, 0)),
    ],
    out_specs=[
        pl.BlockSpec((BR, H), lambda i: (i, 0)),
        pl.BlockSpec((BR, H), lambda i: (i, 0)),
    ],
    out_shape=[
        jax.ShapeDtypeStruct((NPAD, H), jnp.float32),
        jax.ShapeDtypeStruct((NPAD, H), jnp.float32),
    ],
)


def _tc3_body(a0_ref, a1_ref, h1_ref, x0_ref, dinv_ref, w2_ref, g2_ref,
              be2_ref, wr1_ref, br1_ref, gr_ref, ber_ref, wr2_ref, br2_ref,
              o_ref):
    dv = dinv_ref[...]
    h1 = h1_ref[...]
    h = dv * (a0_ref[...] + a1_ref[...]) + (dv * dv) * h1
    t = (1.0 - ALPHA) * h + ALPHA * x0_ref[...]
    u = jnp.dot(t, w2_ref[...], preferred_element_type=jnp.float32)
    h2 = jnp.maximum(_ln(u, g2_ref[...], be2_ref[...]), 0.0)
    z = jnp.dot(h2, wr1_ref[...], preferred_element_type=jnp.float32) + br1_ref[...]
    z = jnp.maximum(_ln(z, gr_ref[...], ber_ref[...]), 0.0)
    o_ref[...] = (jnp.dot(z, wr2_ref[...], preferred_element_type=jnp.float32)
                  + br2_ref[...])


_tc3_call = pl.pallas_call(
    _tc3_body,
    grid=(GRID,),
    in_specs=[
        pl.BlockSpec((BR, H), lambda i: (i, 0)),
        pl.BlockSpec((BR, H), lambda i: (i, 0)),
        pl.BlockSpec((BR, H), lambda i: (i, 0)),
        pl.BlockSpec((BR, H), lambda i: (i, 0)),
        pl.BlockSpec((BR, 1), lambda i: (i, 0)),
        pl.BlockSpec((H, H), lambda i: (0, 0)),
        pl.BlockSpec((1, H), lambda i: (0, 0)),
        pl.BlockSpec((1, H), lambda i: (0, 0)),
        pl.BlockSpec((H, 32), lambda i: (0, 0)),
        pl.BlockSpec((1, 32), lambda i: (0, 0)),
        pl.BlockSpec((1, 32), lambda i: (0, 0)),
        pl.BlockSpec((1, 32), lambda i: (0, 0)),
        pl.BlockSpec((32, OUT), lambda i: (0, 0)),
        pl.BlockSpec((1, OUT), lambda i: (0, 0)),
    ],
    out_specs=pl.BlockSpec((BR, OUT), lambda i: (i, 0)),
    out_shape=jax.ShapeDtypeStruct((NPAD, OUT), jnp.float32),
)


def kernel(x, edge_index, edge_weight, W_in, b_in, W1, g1, be1, W2, g2, be2,
           Wr1, br1, gr, ber, Wr2, br2):
    pad = (TOTC + CMAX) * CHUNK - E
    src_p = jnp.concatenate(
        [edge_index[0], jnp.zeros((pad,), jnp.int32)]).reshape(TOTC + CMAX, CHUNK)
    dst_p = jnp.concatenate(
        [edge_index[1], jnp.full((pad,), N, jnp.int32)]).reshape(TOTC + CMAX, CHUNK)
    x_pad = jnp.pad(x, ((0, NPAD - N), (0, 0)))

    z_deg = jnp.zeros((ROWS_PER_TILE,), jnp.float32)
    ones_c = jnp.ones((CHUNK,), jnp.float32)

    degp = _deg_call(dst_p, z_deg, ones_c)
    x0, y0, dinv = _tc1_call(x_pad, W_in, b_in[None], degp[0][:, None],
                             degp[1][:, None])

    acc1 = _prop_call(y0, src_p, dst_p)
    h1, y1 = _tc2_call(acc1[0], acc1[1], x0, dinv, W1, g1[None], be1[None])

    acc2 = _prop_call(y1, src_p, dst_p)
    out = _tc3_call(acc2[0], acc2[1], h1, x0, dinv, W2, g2[None], be2[None],
                    Wr1, br1[None], gr[None], ber[None], Wr2, br2[None])
    return out[:N]


# speedup vs baseline: 1.0610x; 1.0009x over previous
import jax
import jax.numpy as jnp
from jax import lax
from jax.experimental import pallas as pl
from jax.experimental.pallas import tpu as pltpu
from jax.experimental.pallas import tpu_sc as plsc

N = 10000
NPAD = 10240
E = 160000
DIN = 256
H = 128
OUT = 64
ALPHA = 0.5

NC = 2
NS = 16
NW = NC * NS
CHUNK = 128
CHUNKS_PER_W = 40
TOTC = NW * CHUNKS_PER_W
ROWS_PER_TILE = NPAD // NS
CA = 64
CB = 80 - CA
CMAX = max(CA, CB)

_mesh = plsc.VectorSubcoreMesh(core_axis_name="c", subcore_axis_name="s")


def _deg_body(dst_hbm, zeros_hbm, ones_hbm, out_hbm, idx_v, ones_v, acc_sh, sem):
    c = lax.axis_index("c")
    s = lax.axis_index("s")
    wid = c * NS + s
    pltpu.sync_copy(zeros_hbm, acc_sh.at[pl.ds(s * ROWS_PER_TILE, ROWS_PER_TILE)])
    pltpu.sync_copy(dst_hbm.at[pl.ds(wid * CHUNKS_PER_W, CHUNKS_PER_W)], idx_v)
    pltpu.sync_copy(ones_hbm, ones_v)
    plsc.subcore_barrier()

    @pl.loop(0, CHUNKS_PER_W)
    def _(j):
        pltpu.sync_copy(ones_v, acc_sh.at[idx_v.at[j]], add=True)

    plsc.subcore_barrier()
    pltpu.sync_copy(acc_sh.at[pl.ds(s * ROWS_PER_TILE, ROWS_PER_TILE)],
                    out_hbm.at[c, pl.ds(s * ROWS_PER_TILE, ROWS_PER_TILE)])


_deg_call = pl.kernel(
    _deg_body,
    out_type=jax.ShapeDtypeStruct((NC, NPAD), jnp.float32),
    mesh=_mesh,
    scratch_types=[
        pltpu.VMEM((CHUNKS_PER_W, CHUNK), jnp.int32),
        pltpu.VMEM((CHUNK,), jnp.float32),
        pltpu.VMEM_SHARED((NPAD,), jnp.float32),
        pltpu.SemaphoreType.DMA,
    ],
)


def _prop_pipeline(n, y_hbm, src_v, dst_v, rows0_v, rows1_v, acc_sh,
                   sem0, sem1, ssem0, ssem1):
    def _gather(j, rows, sem):
        pltpu.async_copy(y_hbm.at[src_v.at[j]], rows, sem)

    def _gwait(j, rows, sem):
        pltpu.make_async_copy(y_hbm.at[src_v.at[j]], rows, sem).wait()

    def _scat(j, rows, sem):
        pltpu.async_copy(rows, acc_sh.at[dst_v.at[j]], sem, add=True)

    def _swait(j, rows, sem):
        pltpu.make_async_copy(rows, acc_sh.at[dst_v.at[j]], sem).wait()

    _gather(0, rows0_v, sem0)
    _gather(1, rows1_v, sem1)

    @pl.loop(0, n - 2, step=2)
    def _(j):
        _gwait(j, rows0_v, sem0)
        _scat(j, rows0_v, ssem0)
        _gwait(j + 1, rows1_v, sem1)
        _scat(j + 1, rows1_v, ssem1)
        _swait(j, rows0_v, ssem0)
        _gather(j + 2, rows0_v, sem0)
        _swait(j + 1, rows1_v, ssem1)
        _gather(j + 3, rows1_v, sem1)

    _J = n - 2
    _gwait(_J, rows0_v, sem0)
    _scat(_J, rows0_v, ssem0)
    _gwait(_J + 1, rows1_v, sem1)
    _scat(_J + 1, rows1_v, ssem1)
    _swait(_J, rows0_v, ssem0)
    _swait(_J + 1, rows1_v, ssem1)


def _prop_body(y_hbm, src_hbm, dst_hbm, out_hbm,
               src_v, dst_v, rows0_v, rows1_v, acc_sh, sem0, sem1, ssem0, ssem1):
    c = lax.axis_index("c")
    s = lax.axis_index("s")
    base = jnp.where(c == 0, s * CA, NS * CA + s * CB)

    @pl.loop(0, CHUNK)
    def _(i):
        for k in range(H // 16):
            rows0_v[i, pl.ds(k * 16, 16)] = jnp.zeros((16,), jnp.float32)

    for t in range(ROWS_PER_TILE // CHUNK):
        pltpu.sync_copy(rows0_v,
                        acc_sh.at[pl.ds(s * ROWS_PER_TILE + t * CHUNK, CHUNK)])
    pltpu.sync_copy(src_hbm.at[pl.ds(base, CMAX)], src_v)
    pltpu.sync_copy(dst_hbm.at[pl.ds(base, CMAX)], dst_v)
    plsc.subcore_barrier()

    args = (y_hbm, src_v, dst_v, rows0_v, rows1_v, acc_sh, sem0, sem1,
            ssem0, ssem1)
    if CA == CB:
        _prop_pipeline(CA, *args)
    else:
        @pl.when(c == 0)
        def _():
            _prop_pipeline(CA, *args)

        @pl.when(c == 1)
        def _():
            _prop_pipeline(CB, *args)

    plsc.subcore_barrier()
    pltpu.sync_copy(acc_sh.at[pl.ds(s * ROWS_PER_TILE, ROWS_PER_TILE)],
                    out_hbm.at[c, pl.ds(s * ROWS_PER_TILE, ROWS_PER_TILE)])


_prop_call = pl.kernel(
    _prop_body,
    out_type=jax.ShapeDtypeStruct((NC, NPAD, H), jnp.float32),
    mesh=_mesh,
    scratch_types=[
        pltpu.VMEM((CMAX, CHUNK), jnp.int32),
        pltpu.VMEM((CMAX, CHUNK), jnp.int32),
        pltpu.VMEM((CHUNK, H), jnp.float32),
        pltpu.VMEM((CHUNK, H), jnp.float32),
        pltpu.VMEM_SHARED((NPAD, H), jnp.float32),
        pltpu.SemaphoreType.DMA,
        pltpu.SemaphoreType.DMA,
        pltpu.SemaphoreType.DMA,
        pltpu.SemaphoreType.DMA,
    ],
)


BR = 512
GRID = NPAD // BR


def _ln(u, g, b):
    mu = jnp.mean(u, axis=-1, keepdims=True)
    var = jnp.mean((u - mu) ** 2, axis=-1, keepdims=True)
    return (u - mu) * lax.rsqrt(var + 1e-5) * g + b


def _tc1_body(x_ref, wi_ref, bi_ref, dega_ref, degb_ref,
              x0_ref, y0_ref, dinv_ref):
    x0 = jnp.maximum(
        jnp.dot(x_ref[...], wi_ref[...], preferred_element_type=jnp.float32)
        + bi_ref[...], 0.0)
    deg = dega_ref[...] + degb_ref[...] + 1.0
    dinv = lax.rsqrt(deg)
    x0_ref[...] = x0
    y0_ref[...] = dinv * x0
    dinv_ref[...] = dinv


_tc1_call = pl.pallas_call(
    _tc1_body,
    grid=(GRID,),
    in_specs=[
        pl.BlockSpec((BR, DIN), lambda i: (i, 0)),
        pl.BlockSpec((DIN, H), lambda i: (0, 0)),
        pl.BlockSpec((1, H), lambda i: (0, 0)),
        pl.BlockSpec((BR, 1), lambda i: (i, 0)),
        pl.BlockSpec((BR, 1), lambda i: (i, 0)),
    ],
    out_specs=[
        pl.BlockSpec((BR, H), lambda i: (i, 0)),
        pl.BlockSpec((BR, H), lambda i: (i, 0)),
        pl.BlockSpec((BR, 1), lambda i: (i, 0)),
    ],
    out_shape=[
        jax.ShapeDtypeStruct((NPAD, H), jnp.float32),
        jax.ShapeDtypeStruct((NPAD, H), jnp.float32),
        jax.ShapeDtypeStruct((NPAD, 1), jnp.float32),
    ],
)


def _tc2_body(a0_ref, a1_ref, x0_ref, dinv_ref, w1_ref, g1_ref, be1_ref,
              h1_ref, y1_ref):
    dv = dinv_ref[...]
    x0 = x0_ref[...]
    h = dv * (a0_ref[...] + a1_ref[...]) + (dv * dv) * x0
    t = (1.0 - ALPHA) * h + ALPHA * x0
    u = jnp.dot(t, w1_ref[...], preferred_element_type=jnp.float32)
    h1 = jnp.maximum(_ln(u, g1_ref[...], be1_ref[...]), 0.0)
    h1_ref[...] = h1
    y1_ref[...] = dv * h1


_tc2_call = pl.pallas_call(
    _tc2_body,
    grid=(GRID,),
    in_specs=[
        pl.BlockSpec((BR, H), lambda i: (i, 0)),
        pl.BlockSpec((BR, H), lambda i: (i, 0)),
        pl.BlockSpec((BR, H), lambda i: (i, 0)),
        pl.BlockSpec((BR, 1), lambda i: (i, 0)),
        pl.BlockSpec((H, H), lambda i: (0, 0)),
        pl.BlockSpec((1, H), lambda i: (0, 0)),
        pl.BlockSpec((1, H), lambda i: (0, 0)),
    ],
    out_specs=[
        pl.BlockSpec((BR, H), lambda i: (i, 0)),
        pl.BlockSpec((BR, H), lambda i: (i, 0)),
    ],
    out_shape=[
        jax.ShapeDtypeStruct((NPAD, H), jnp.float32),
        jax.ShapeDtypeStruct((NPAD, H), jnp.float32),
    ],
)


def _tc3_body(a0_ref, a1_ref, h1_ref, x0_ref, dinv_ref, w2_ref, g2_ref,
              be2_ref, wr1_ref, br1_ref, gr_ref, ber_ref, wr2_ref, br2_ref,
              o_ref):
    dv = dinv_ref[...]
    h1 = h1_ref[...]
    h = dv * (a0_ref[...] + a1_ref[...]) + (dv * dv) * h1
    t = (1.0 - ALPHA) * h + ALPHA * x0_ref[...]
    u = jnp.dot(t, w2_ref[...], preferred_element_type=jnp.float32)
    h2 = jnp.maximum(_ln(u, g2_ref[...], be2_ref[...]), 0.0)
    z = jnp.dot(h2, wr1_ref[...], preferred_element_type=jnp.float32) + br1_ref[...]
    z = jnp.maximum(_ln(z, gr_ref[...], ber_ref[...]), 0.0)
    o_ref[...] = (jnp.dot(z, wr2_ref[...], preferred_element_type=jnp.float32)
                  + br2_ref[...])


_tc3_call = pl.pallas_call(
    _tc3_body,
    grid=(GRID,),
    in_specs=[
        pl.BlockSpec((BR, H), lambda i: (i, 0)),
        pl.BlockSpec((BR, H), lambda i: (i, 0)),
        pl.BlockSpec((BR, H), lambda i: (i, 0)),
        pl.BlockSpec((BR, H), lambda i: (i, 0)),
        pl.BlockSpec((BR, 1), lambda i: (i, 0)),
        pl.BlockSpec((H, H), lambda i: (0, 0)),
        pl.BlockSpec((1, H), lambda i: (0, 0)),
        pl.BlockSpec((1, H), lambda i: (0, 0)),
        pl.BlockSpec((H, 32), lambda i: (0, 0)),
        pl.BlockSpec((1, 32), lambda i: (0, 0)),
        pl.BlockSpec((1, 32), lambda i: (0, 0)),
        pl.BlockSpec((1, 32), lambda i: (0, 0)),
        pl.BlockSpec((32, OUT), lambda i: (0, 0)),
        pl.BlockSpec((1, OUT), lambda i: (0, 0)),
    ],
    out_specs=pl.BlockSpec((BR, OUT), lambda i: (i, 0)),
    out_shape=jax.ShapeDtypeStruct((NPAD, OUT), jnp.float32),
)


def kernel(x, edge_index, edge_weight, W_in, b_in, W1, g1, be1, W2, g2, be2,
           Wr1, br1, gr, ber, Wr2, br2):
    pad = (TOTC + CMAX) * CHUNK - E
    src_p = jnp.concatenate(
        [edge_index[0], jnp.zeros((pad,), jnp.int32)]).reshape(TOTC + CMAX, CHUNK)
    dst_p = jnp.concatenate(
        [edge_index[1], jnp.full((pad,), N, jnp.int32)]).reshape(TOTC + CMAX, CHUNK)
    x_pad = jnp.pad(x, ((0, NPAD - N), (0, 0)))

    z_deg = jnp.zeros((ROWS_PER_TILE,), jnp.float32)
    ones_c = jnp.ones((CHUNK,), jnp.float32)

    degp = _deg_call(dst_p, z_deg, ones_c)
    x0, y0, dinv = _tc1_call(x_pad, W_in, b_in[None], degp[0][:, None],
                             degp[1][:, None])

    acc1 = _prop_call(y0, src_p, dst_p)
    h1, y1 = _tc2_call(acc1[0], acc1[1], x0, dinv, W1, g1[None], be1[None])

    acc2 = _prop_call(y1, src_p, dst_p)
    out = _tc3_call(acc2[0], acc2[1], h1, x0, dinv, W2, g2[None], be2[None],
                    Wr1, br1[None], gr[None], ber[None], Wr2, br2[None])
    return out[:N]
